# baseline (speedup 1.0000x reference)
import functools

import jax
import jax.numpy as jnp
from jax import lax
from jax.experimental import pallas as pl
from jax.experimental.pallas import tpu as pltpu
from jax.experimental.pallas import tpu_sc as plsc

F32 = jnp.float32
I32 = jnp.int32

N0 = 10000
E0 = 160000
NW = 32
CHUNK = 128
EPAD = 163840
NCH = EPAD // (NW * CHUNK)
EPW = EPAD // NW
EPADT = EPAD + 128
ERWS = EPADT // CHUNK

NS_ = [10000, 5000, 2500, 1250, 1000]
RS_ = [10240, 5120, 2560, 1280, 1024]
KS_ = [5000, 2500, 1250, 1000, 800]
KPAD_ = [5120, 2560, 1280, 1024, 1024]
PSORT_ = [16384, 8192, 4096, 2048, 1024]
DIN_ = [128, 256, 128, 64, 64]
DOUT_ = [256, 128, 64, 64, 64]
BR_ = [1024, 1024, 2560, 1280, 1024]
MODE_ = ["zu2", "zu4", "zu2", "comb", "comb"]


def _mesh():
    return plsc.VectorSubcoreMesh(core_axis_name="c", subcore_axis_name="s",
                                  num_cores=2, num_subcores=16)


def _sc_segsum(table, src, dsteff, counts, nrows):
    d = 128
    free_words = 2040000 - nrows * 128 - 16 * (2 * NCH * CHUNK) - 16 * 1024
    nbuf = max(2, min(6, free_words // (16 * CHUNK * 128)))
    rows_pt = nrows // 16

    @functools.partial(
        pl.kernel, mesh=_mesh(),
        out_type=jax.ShapeDtypeStruct((2, nrows, d), F32),
        scratch_types=[
            pltpu.VMEM_SHARED((nrows, d), F32),
            pltpu.VMEM((16,), I32),
            pltpu.VMEM((NCH, CHUNK), I32),
            pltpu.VMEM((NCH, CHUNK), I32),
        ] + [pltpu.VMEM((CHUNK, d), F32)] * nbuf
          + [pltpu.SemaphoreType.DMA] * (2 * nbuf)
          + [pltpu.SemaphoreType.DMA])
    def k(tab, srch, dsth, cnth, zer, out, acc, cntv, sall, dall, *rest):
        rows = rest[:nbuf]
        gsem = rest[nbuf:2 * nbuf]
        ssem = rest[2 * nbuf:3 * nbuf]
        isem = rest[3 * nbuf]
        ci = lax.axis_index("c")
        si = lax.axis_index("s")
        w = si * 2 + ci
        row0 = w * NCH
        pltpu.sync_copy(cnth.at[w], cntv)
        cnt = cntv[...][0]
        pltpu.async_copy(srch.at[pl.ds(row0, NCH)], sall, isem)
        pltpu.async_copy(dsth.at[pl.ds(row0, NCH)], dall, isem)
        pltpu.sync_copy(zer, acc.at[pl.ds(si * rows_pt, rows_pt)])
        pltpu.make_async_copy(srch.at[pl.ds(row0, NCH)], sall, isem).wait()
        pltpu.make_async_copy(dsth.at[pl.ds(row0, NCH)], dall, isem).wait()
        plsc.subcore_barrier()

        for b in range(nbuf):
            @pl.when(b < cnt)
            def _(b=b):
                pltpu.async_copy(tab.at[sall.at[b]], rows[b], gsem[b])

        def group(q, carry):
            j0 = q * nbuf
            for b in range(nbuf):
                @pl.when(j0 + b < cnt)
                def _(b=b, j=j0 + b):
                    pltpu.make_async_copy(tab.at[sall.at[j]], rows[b],
                                          gsem[b]).wait()
                    pltpu.async_copy(rows[b], acc.at[dall.at[j]], ssem[b],
                                     add=True)
            for b in range(nbuf):
                @pl.when(j0 + b + nbuf < cnt)
                def _(b=b, j=j0 + b):
                    pltpu.make_async_copy(rows[b], acc.at[dall.at[j]],
                                          ssem[b]).wait()
                    pltpu.async_copy(tab.at[sall.at[j + nbuf]], rows[b],
                                     gsem[b])
            return carry

        lax.fori_loop(0, (cnt + nbuf - 1) // nbuf, group, 0)
        for b in range(nbuf):
            @pl.when((cnt >= nbuf) | (b < cnt))
            def _(b=b):
                pltpu.make_async_copy(rows[b], acc.at[dall.at[0]],
                                      ssem[b]).wait()
        plsc.subcore_barrier()
        pltpu.sync_copy(acc.at[pl.ds(si * rows_pt, rows_pt)],
                        out.at[ci, pl.ds(si * rows_pt, rows_pt)])

    return k(table, src, dsteff, counts, jnp.zeros((rows_pt, d), F32))


def _sc_relabel(new_idx, srcr, de, counts, dummy_old, k_new):
    nbuf = 4
    osd = jax.ShapeDtypeStruct((ERWS, CHUNK), I32)

    @functools.partial(
        pl.kernel, mesh=_mesh(),
        out_type=(osd, osd),
        scratch_types=[pltpu.VMEM((16,), I32)]
        + [pltpu.VMEM((CHUNK,), I32)] * (6 * nbuf)
        + [pltpu.SemaphoreType.DMA] * (3 * nbuf))
    def k(nix_h, src_h, de_h, cnt_h, os_h, oe_h, cntv, *rest):
        bs = rest[0:nbuf]
        be = rest[nbuf:2 * nbuf]
        ns = rest[2 * nbuf:3 * nbuf]
        ne = rest[3 * nbuf:4 * nbuf]
        cs = rest[4 * nbuf:5 * nbuf]
        ce = rest[5 * nbuf:6 * nbuf]
        sems = rest[6 * nbuf:]
        la = sems[0:nbuf]
        lb = sems[nbuf:2 * nbuf]
        lc = sems[2 * nbuf:3 * nbuf]
        ci = lax.axis_index("c")
        si = lax.axis_index("s")
        w = si * 2 + ci
        row0 = w * NCH
        pltpu.sync_copy(cnt_h.at[w], cntv)
        cnt = cntv[...][0]

        def fire_loads(j, b):
            pltpu.async_copy(src_h.at[row0 + j], bs[b], la[b])
            pltpu.async_copy(de_h.at[row0 + j], be[b], la[b])

        for b in range(nbuf):
            @pl.when(b < cnt)
            def _(b=b):
                fire_loads(b, b)

        def group(q, carry):
            j0 = q * nbuf
            for b in range(nbuf):
                @pl.when(j0 + b < cnt)
                def _(b=b, j=j0 + b):
                    pltpu.make_async_copy(src_h.at[row0 + j], bs[b],
                                          la[b]).wait()
                    pltpu.make_async_copy(de_h.at[row0 + j], be[b],
                                          la[b]).wait()
                    pltpu.async_copy(nix_h.at[bs[b]], ns[b], lb[b])
                    pltpu.async_copy(nix_h.at[be[b]], ne[b], lb[b])
            for b in range(nbuf):
                @pl.when(j0 + b < cnt)
                def _(b=b, j=j0 + b):
                    pltpu.make_async_copy(nix_h.at[bs[b]], ns[b], lb[b]).wait()
                    pltpu.make_async_copy(nix_h.at[be[b]], ne[b], lb[b]).wait()

                    @pl.when(j > nbuf - 1)
                    def _():
                        pltpu.make_async_copy(cs[b], os_h.at[row0], lc[b]).wait()
                        pltpu.make_async_copy(ce[b], oe_h.at[row0], lc[b]).wait()

                    def grp(g, c2):
                        sl = pl.ds(g * 16, 16)
                        val = be[b][sl] != dummy_old
                        nsv = ns[b][sl]
                        nev = ne[b][sl]
                        v2 = val & (nsv >= 0) & (nev >= 0)
                        cs[b][sl] = jnp.where(v2, nsv, 0)
                        ce[b][sl] = jnp.where(v2, nev, k_new)
                        return c2

                    lax.fori_loop(0, CHUNK // 16, grp, 0)
                    pltpu.async_copy(cs[b], os_h.at[row0 + j], lc[b])
                    pltpu.async_copy(ce[b], oe_h.at[row0 + j], lc[b])

                    @pl.when(j + nbuf < cnt)
                    def _(j=j, b=b):
                        fire_loads(j + nbuf, b)
            return carry

        lax.fori_loop(0, (cnt + nbuf - 1) // nbuf, group, 0)
        for b in range(nbuf):
            @pl.when((cnt >= nbuf) | (b < cnt))
            def _(b=b):
                pltpu.make_async_copy(cs[b], os_h.at[row0], lc[b]).wait()
                pltpu.make_async_copy(ce[b], oe_h.at[row0], lc[b]).wait()

    return k(new_idx, srcr, de, counts)


def _tc_positions(de, counts, dummy):
    r = EPAD // 128
    gr = EPW // 128

    def body(der, cnr, po, co):
        d2 = der[...]
        rowid0 = lax.broadcasted_iota(I32, (r, 128), 0)
        crow = cnr[...][:, 0:1].astype(F32)
        sel = (lax.broadcasted_iota(I32, (r, NW), 0) // gr
               == lax.broadcasted_iota(I32, (r, NW), 1)).astype(F32)
        lim = jnp.dot(sel, crow, preferred_element_type=F32).astype(I32)
        covered = (rowid0 % gr) < lim
        d2 = jnp.where(covered, d2, dummy)
        v = (d2 != dummy).astype(I32)
        colid = lax.broadcasted_iota(I32, (r, 128), 1)
        rowid = lax.broadcasted_iota(I32, (r, 128), 0)
        p = v
        dd = 1
        while dd < 128:
            p = p + jnp.where(colid >= dd, pltpu.roll(p, dd, axis=1), 0)
            dd *= 2
        rs = p[:, 127:128]
        rr = rowid[:, 0:1] % gr
        ex = jnp.where(rr >= 1, pltpu.roll(rs, 1, axis=0), 0)
        dd = 1
        while dd < gr:
            ex = ex + jnp.where(rr >= dd, pltpu.roll(ex, dd, axis=0), 0)
            dd *= 2
        posl = (p - v) + ex
        pos = (rowid // gr) * EPW + posl
        pos = jnp.where(d2 != dummy, pos, EPAD)
        po[...] = pos
        sel2 = (lax.broadcasted_iota(I32, (NW, r), 0)
                == lax.broadcasted_iota(I32, (NW, r), 1) // gr).astype(F32)
        tot = jnp.dot(sel2, rs.astype(F32),
                      preferred_element_type=F32).astype(I32)
        co[...] = jnp.broadcast_to((tot + 127) // 128, (NW, 16))

    return pl.pallas_call(
        body,
        out_shape=(jax.ShapeDtypeStruct((r, 128), I32),
                   jax.ShapeDtypeStruct((NW, 16), I32)))(de, counts)


def _sc_compact(src2, de2, pos, cnt_in, cnt_out, k_new):
    nbuf = 4
    osd = jax.ShapeDtypeStruct((EPADT,), I32)

    @functools.partial(
        pl.kernel, mesh=_mesh(),
        out_type=(osd, osd),
        scratch_types=[pltpu.VMEM((16,), I32), pltpu.VMEM((16,), I32)]
        + [pltpu.VMEM((CHUNK,), I32)] * (3 * nbuf + 2)
        + [pltpu.SemaphoreType.DMA] * (2 * nbuf))
    def k(src_h, de_h, pos_h, ci_h, co_h, os_h, oe_h, civ, cov, *rest):
        sb = rest[0:nbuf]
        eb = rest[nbuf:2 * nbuf]
        pb = rest[2 * nbuf:3 * nbuf]
        dums = rest[3 * nbuf]
        dume = rest[3 * nbuf + 1]
        la = rest[3 * nbuf + 2:3 * nbuf + 2 + nbuf]
        lb = rest[3 * nbuf + 2 + nbuf:]
        ci = lax.axis_index("c")
        si = lax.axis_index("s")
        w = si * 2 + ci
        row0 = w * NCH
        pltpu.sync_copy(ci_h.at[w], civ)
        pltpu.sync_copy(co_h.at[w], cov)
        cin = civ[...][0]
        cout = cov[...][0]

        def fill(g, carry):
            sl = pl.ds(g * 16, 16)
            dums[sl] = jnp.zeros((16,), I32)
            dume[sl] = jnp.full((16,), k_new, I32)
            return carry

        lax.fori_loop(0, CHUNK // 16, fill, 0)

        @pl.when(cout > 0)
        def _():
            tb = (row0 + cout - 1) * CHUNK
            pltpu.sync_copy(dums, os_h.at[pl.ds(tb, CHUNK)])
            pltpu.sync_copy(dume, oe_h.at[pl.ds(tb, CHUNK)])

        def fire_loads(j, b):
            pltpu.async_copy(src_h.at[row0 + j], sb[b], la[b])
            pltpu.async_copy(de_h.at[row0 + j], eb[b], la[b])
            pltpu.async_copy(pos_h.at[row0 + j], pb[b], la[b])

        for b in range(nbuf):
            @pl.when(b < cin)
            def _(b=b):
                fire_loads(b, b)

        def group(q, carry):
            j0 = q * nbuf
            for b in range(nbuf):
                @pl.when(j0 + b < cin)
                def _(b=b, j=j0 + b):
                    pltpu.make_async_copy(src_h.at[row0], sb[b], la[b]).wait()
                    pltpu.make_async_copy(de_h.at[row0], eb[b], la[b]).wait()
                    pltpu.make_async_copy(pos_h.at[row0], pb[b], la[b]).wait()
                    pltpu.async_copy(sb[b], os_h.at[pb[b]], lb[b])
                    pltpu.async_copy(eb[b], oe_h.at[pb[b]], lb[b])
            for b in range(nbuf):
                @pl.when(j0 + b + nbuf < cin)
                def _(b=b, j=j0 + b):
                    pltpu.make_async_copy(sb[b], os_h.at[pb[b]], lb[b]).wait()
                    pltpu.make_async_copy(eb[b], oe_h.at[pb[b]], lb[b]).wait()
                    fire_loads(j + nbuf, b)
            return carry

        lax.fori_loop(0, (cin + nbuf - 1) // nbuf, group, 0)
        for b in range(nbuf):
            @pl.when((cin >= nbuf) | (b < cin))
            def _(b=b):
                pltpu.make_async_copy(sb[b], os_h.at[pb[b]], lb[b]).wait()
                pltpu.make_async_copy(eb[b], oe_h.at[pb[b]], lb[b]).wait()

    return k(src2, de2, pos, cnt_in, cnt_out)


def _sc_gather(table, perm, kpad):
    d = 128
    rows_pt = kpad // NW
    gc = 80 if rows_pt % 80 == 0 else rows_pt
    nchunks = rows_pt // gc

    @functools.partial(
        pl.kernel, mesh=_mesh(),
        out_type=jax.ShapeDtypeStruct((kpad, d), F32),
        scratch_types=[pltpu.VMEM((rows_pt,), I32)]
        + [pltpu.VMEM((gc, d), F32)] * 2
        + [pltpu.SemaphoreType.DMA] * 3)
    def k(tab, perm_h, out, pidx, r0, r1, g0, g1, so):
        ci = lax.axis_index("c")
        si = lax.axis_index("s")
        w = si * 2 + ci
        base = w * rows_pt
        pltpu.sync_copy(perm_h.at[pl.ds(base, rows_pt)], pidx)
        bufs = [r0, r1]
        sems = [g0, g1]
        for c in range(nchunks):
            pltpu.async_copy(tab.at[pidx.at[pl.ds(c * gc, gc)]],
                             bufs[c % 2], sems[c % 2])
        for c in range(nchunks):
            pltpu.make_async_copy(tab.at[pidx.at[pl.ds(c * gc, gc)]],
                                  bufs[c % 2], sems[c % 2]).wait()
            pltpu.async_copy(bufs[c % 2], out.at[pl.ds(base + c * gc, gc)], so)
        for c in range(nchunks):
            pltpu.make_async_copy(r0, out.at[pl.ds(base, gc)], so).wait()

    return k(table, perm)


def _tc_scale(xg, scv, kpad):
    sc3d = scv.reshape(kpad // 128, 1, 128)
    g = kpad // 128

    def body(xr, sr, o):
        ident = (lax.broadcasted_iota(I32, (128, 128), 0)
                 == lax.broadcasted_iota(I32, (128, 128), 1)).astype(F32)
        col = lax.dot_general(ident, sr[...][0], (((1,), (1,)), ((), ())),
                              preferred_element_type=F32)
        o[...] = xr[...] * col

    return pl.pallas_call(
        body, grid=(g,),
        in_specs=[pl.BlockSpec((128, 128), lambda i: (i, 0)),
                  pl.BlockSpec((1, 1, 128), lambda i: (i, 0, 0))],
        out_specs=pl.BlockSpec((128, 128), lambda i: (i, 0)),
        out_shape=jax.ShapeDtypeStruct((kpad, 128), F32))(xg, sc3d)


def _tc_colmax(x, nrows, din, br):
    g = nrows // br
    xw = x.shape[1]

    def body(xr, o):
        y = jnp.maximum(xr[...][:, :din], 0.0) + 1e-7
        o[...] = jnp.max(y, axis=0, keepdims=True)[None]

    return pl.pallas_call(
        body, grid=(g,),
        in_specs=[pl.BlockSpec((br, xw), lambda i: (i, 0))],
        out_specs=pl.BlockSpec((1, 1, din), lambda i: (i, 0, 0)),
        out_shape=jax.ShapeDtypeStruct((g, 1, din), F32))(x)


def _tc_zu(x, part, nrows, din, br, mode):
    g = nrows // br
    xw = x.shape[1]
    gp = part.shape[0]
    nslab = {"zu2": 2, "zu4": 4, "comb": 1}[mode]

    def body(xr, pr, *outs):
        m = jnp.max(pr[...][:, 0, :], axis=0, keepdims=True)
        y = jnp.maximum(xr[...][:, :din], 0.0) + 1e-7
        z = jnp.exp(y - m)
        u = z * y
        if mode == "zu2":
            outs[0][...] = z
            outs[1][...] = u
        elif mode == "zu4":
            outs[0][...] = z[:, :128]
            outs[1][...] = z[:, 128:]
            outs[2][...] = u[:, :128]
            outs[3][...] = u[:, 128:]
        else:
            outs[0][...] = jnp.concatenate([z, u], axis=1)

    sd = jax.ShapeDtypeStruct((nrows, 128), F32)
    bs = pl.BlockSpec((br, 128), lambda i: (i, 0))
    return pl.pallas_call(
        body, grid=(g,),
        in_specs=[pl.BlockSpec((br, xw), lambda i: (i, 0)),
                  pl.BlockSpec((gp, 1, din), lambda i: (0, 0, 0))],
        out_specs=(bs,) * nslab, out_shape=(sd,) * nslab)(x, part)


def _tc_mlp(accs, x, w1, b1, gmm, bet, w2, b2, nrows, din, dout, br, mode):
    g = nrows // br
    hm = 2 * din
    xw = x.shape[1]
    nacc = len(accs)
    nout = 2 if dout == 256 else 1

    def body(*refs):
        ar = refs[:nacc]
        xr = refs[nacc]
        w1r, b1r, gr, btr, w2r, b2r = refs[nacc + 1:nacc + 7]
        outs = refs[nacc + 7:]
        sums = [a[...][0] + a[...][1] for a in ar]
        if mode == "zu2":
            s, t = sums
        elif mode == "zu4":
            s = jnp.concatenate(sums[:2], axis=1)
            t = jnp.concatenate(sums[2:], axis=1)
        else:
            s = sums[0][:, :din]
            t = sums[0][:, din:2 * din]
        aggr = t / (s + 1e-16)
        o = aggr + xr[...][:, :din]
        h1 = jnp.dot(o, w1r[...], preferred_element_type=F32) + b1r[...]
        h1 = jnp.maximum(h1 * gr[...] + btr[...], 0.0)
        h = jnp.dot(h1, w2r[...], preferred_element_type=F32) + b2r[...]
        h = jnp.maximum(h, 0.0)
        if dout == 256:
            outs[0][...] = h[:, :128]
            outs[1][...] = h[:, 128:]
        elif dout == 64:
            outs[0][...] = jnp.concatenate(
                [h, jnp.zeros((h.shape[0], 64), F32)], axis=1)
        else:
            outs[0][...] = h

    wspecs = [pl.BlockSpec((din, hm), lambda i: (0, 0)),
              pl.BlockSpec((1, hm), lambda i: (0, 0)),
              pl.BlockSpec((1, hm), lambda i: (0, 0)),
              pl.BlockSpec((1, hm), lambda i: (0, 0)),
              pl.BlockSpec((hm, dout), lambda i: (0, 0)),
              pl.BlockSpec((1, dout), lambda i: (0, 0))]
    aspec = pl.BlockSpec((2, br, 128), lambda i: (0, i, 0))
    in_specs = ([aspec] * nacc
                + [pl.BlockSpec((br, xw), lambda i: (i, 0))] + wspecs)
    sd = jax.ShapeDtypeStruct((nrows, 128), F32)
    bs = pl.BlockSpec((br, 128), lambda i: (i, 0))
    return pl.pallas_call(
        body, grid=(g,), in_specs=in_specs,
        out_specs=(bs,) * nout, out_shape=(sd,) * nout)(
            *accs, x, w1, b1, gmm, bet, w2, b2)


def _tc_score(naccs, hs, wrels, wroots, brel, nrows, n, br):
    g = nrows // br
    np_ = len(naccs)

    def body(*refs):
        nas = refs[:np_]
        hhs = refs[np_:2 * np_]
        wre = refs[2 * np_:3 * np_]
        wro = refs[3 * np_:4 * np_]
        brr = refs[4 * np_]
        o = refs[-1]
        sc = jnp.zeros((nas[0][...].shape[1],), F32) + brr[0, 0]
        for a, w in zip(nas, wre):
            sc = sc + jnp.sum((a[...][0] + a[...][1]) * w[...], axis=1)
        for h, w in zip(hhs, wro):
            sc = sc + jnp.sum(h[...] * w[...], axis=1)
        sc = jnp.tanh(sc)
        row = pl.program_id(0) * br + lax.iota(I32, br)
        sc = jnp.where(row < n, sc, -2.0)
        o[...] = sc.reshape(br // 128, 128)

    in_specs = ([pl.BlockSpec((2, br, 128), lambda i: (0, i, 0))] * np_
                + [pl.BlockSpec((br, 128), lambda i: (i, 0))] * np_
                + [pl.BlockSpec((1, 128), lambda i: (0, 0))] * (2 * np_)
                + [pl.BlockSpec((1, 1), lambda i: (0, 0))])
    return pl.pallas_call(
        body, grid=(g,), in_specs=in_specs,
        out_specs=pl.BlockSpec((br // 128, 128), lambda i: (i, 0)),
        out_shape=jax.ShapeDtypeStruct((nrows // 128, 128), F32))(
            *naccs, *hs, *wrels, *wroots, brel)


def _sortnet(key, idx, p):
    r = p // 128
    rowid = lax.broadcasted_iota(I32, (r, 128), 0)
    colid = lax.broadcasted_iota(I32, (r, 128), 1)
    gid = rowid * 128 + colid
    k = 2
    while k <= p:
        j = k // 2
        while j >= 1:
            if j >= 128:
                jr = j // 128
                pk_m = pltpu.roll(key, r - jr, axis=0)
                pk_p = pltpu.roll(key, jr, axis=0)
                pi_m = pltpu.roll(idx, r - jr, axis=0)
                pi_p = pltpu.roll(idx, jr, axis=0)
            else:
                pk_m = pltpu.roll(key, 128 - j, axis=1)
                pk_p = pltpu.roll(key, j, axis=1)
                pi_m = pltpu.roll(idx, 128 - j, axis=1)
                pi_p = pltpu.roll(idx, j, axis=1)
            low = (gid & j) == 0
            pk = jnp.where(low, pk_m, pk_p)
            pi = jnp.where(low, pi_m, pi_p)
            up = (gid & k) == 0
            p_less = (pk < key) | ((pk == key) & (pi < idx))
            take_min = low == up
            key = jnp.where(take_min, jnp.where(p_less, pk, key),
                            jnp.where(p_less, key, pk))
            idx = jnp.where(take_min, jnp.where(p_less, pi, idx),
                            jnp.where(p_less, idx, pi))
            j //= 2
        k *= 2
    return key, idx


def _tc_sort(score, nrows, n, kk, kpad, p):
    rs = nrows // 128
    rp = p // 128

    def body(sref, sco, pco, nio):
        sc2 = sref[...]
        if rp > rs:
            pad = jnp.full((rp - rs, 128), 2.0, F32)
            key = jnp.concatenate([-sc2, pad], axis=0)
        else:
            key = -sc2
        rowid = lax.broadcasted_iota(I32, (rp, 128), 0)
        colid = lax.broadcasted_iota(I32, (rp, 128), 1)
        gid = rowid * 128 + colid
        k1, i1 = _sortnet(key, gid, p)
        sel = gid < kk
        sco[...] = jnp.where(sel, -k1, 0.0)[:kpad // 128].reshape(kpad)
        pco[...] = jnp.where(sel, i1, 0)[:kpad // 128].reshape(kpad)
        pay = jnp.where(sel, gid, -1)
        _, p2 = _sortnet(i1, pay, p)
        nio[...] = p2[:rs].reshape(nrows)

    return pl.pallas_call(
        body,
        out_shape=(jax.ShapeDtypeStruct((kpad,), F32),
                   jax.ShapeDtypeStruct((kpad,), I32),
                   jax.ShapeDtypeStruct((nrows,), I32)))(score)


def _tc_readout(xn, kpad, kk):
    def body(xr, o):
        v = xr[...][:, :64]
        row = lax.broadcasted_iota(I32, (kpad, 64), 0)
        msk = row < kk
        mx = jnp.max(jnp.where(msk, v, -3.4e38), axis=0, keepdims=True)
        mn = jnp.sum(jnp.where(msk, v, 0.0), axis=0, keepdims=True) / kk
        o[...] = jnp.concatenate([mx, mn], axis=1)

    return pl.pallas_call(
        body, out_shape=jax.ShapeDtypeStruct((1, 128), F32))(xn)


def _tc_head(r3, r4, r5, w1, b1, w2, b2, w3, b3):
    def body(a, b, c, w1r, b1r, w2r, b2r, w3r, b3r, o):
        z = a[...] + b[...] + c[...]
        z = jnp.maximum(jnp.dot(z, w1r[...], preferred_element_type=F32)
                        + b1r[...], 0.0)
        z = jnp.maximum(jnp.dot(z, w2r[...], preferred_element_type=F32)
                        + b2r[...], 0.0)
        z = jnp.dot(z, w3r[...], preferred_element_type=F32) + b3r[...]
        m = jnp.max(z, axis=1, keepdims=True)
        o[...] = z - m - jnp.log(jnp.sum(jnp.exp(z - m), axis=1, keepdims=True))

    return pl.pallas_call(
        body, out_shape=jax.ShapeDtypeStruct((1, 10), F32))(
            r3, r4, r5, w1, b1, w2, b2, w3, b3)


def _pad_slab(w):
    c = w.shape[1]
    if c % 128 != 0:
        w = jnp.pad(w, ((0, 0), (0, 128 - c % 128)))
    return [w[:, j * 128:(j + 1) * 128] for j in range(w.shape[1] // 128)]


def kernel(x, edge_index, edge_attr, batch, params):
    del edge_attr, batch
    src = edge_index[0].astype(I32)
    dst = edge_index[1].astype(I32)
    padn = EPADT - E0
    srcr = jnp.concatenate([src, jnp.zeros((padn,), I32)]).reshape(ERWS, CHUNK)
    der = jnp.concatenate(
        [dst, jnp.full((padn,), N0, I32)]).reshape(ERWS, CHUNK)
    counts = jnp.full((NW, 16), NCH, I32).at[NW - 1].set(
        (E0 - (NW - 1) * EPW + CHUNK - 1) // CHUNK)

    xslabs = [jnp.pad(x, ((0, RS_[0] - N0), (0, 0)))]
    readouts = []
    bninv = 1.0 / jnp.sqrt(jnp.float32(1.0 + 1e-5))

    for i in range(5):
        nrows, n, kk, kpad = RS_[i], NS_[i], KS_[i], KPAD_[i]
        din, dout, br, mode = DIN_[i], DOUT_[i], BR_[i], MODE_[i]
        p = params['conv%d' % (i + 1)]
        pp = params['pool%d' % (i + 1)]
        gmm = (p['gamma'] * bninv).reshape(1, -1)
        bet = p['beta'].reshape(1, -1)
        b1 = p['b1'].reshape(1, -1)
        b2 = p['b2'].reshape(1, -1)

        xin = xslabs[0] if len(xslabs) == 1 else jnp.concatenate(xslabs, 1)
        part = _tc_colmax(xin, nrows, din, br)
        slabs = _tc_zu(xin, part, nrows, din, br, mode)
        accs = [_sc_segsum(s, srcr, der, counts, nrows) for s in slabs]
        hs = _tc_mlp(accs, xin, p['W1'], b1, gmm, bet, p['W2'], b2,
                     nrows, din, dout, br, mode)

        naccs = [_sc_segsum(h, srcr, der, counts, nrows) for h in hs]
        wrels = _pad_slab(pp['Wrel'].reshape(1, -1))
        wroots = _pad_slab(pp['Wroot'].reshape(1, -1))
        brel = pp['brel'].reshape(1, 1)
        score = _tc_score(naccs, hs, wrels, wroots, brel, nrows, n, br)
        scv, perm, new_idx = _tc_sort(score, nrows, n, kk, kpad, PSORT_[i])

        xslabs = [_tc_scale(_sc_gather(h, perm, kpad), scv, kpad)
                  for h in hs]
        if i < 4:
            ns2, ne2 = _sc_relabel(new_idx, srcr, der, counts, n, kk)
            pos, counts2 = _tc_positions(ne2[:EPAD // CHUNK], counts, kk)
            so, eo = _sc_compact(ns2, ne2, pos, counts, counts2, kk)
            srcr = so.reshape(ERWS, CHUNK)
            der = eo.reshape(ERWS, CHUNK)
            counts = counts2
        if i >= 2:
            readouts.append(_tc_readout(xslabs[0], kpad, kk))

    lp = params['lin1']
    lq = params['lin2']
    lr = params['lin3']
    return _tc_head(readouts[0], readouts[1], readouts[2],
                    lp['W'], lp['b'].reshape(1, -1),
                    lq['W'], lq['b'].reshape(1, -1),
                    lr['W'], lr['b'].reshape(1, -1))
